# baseline (speedup 1.0000x reference)
import functools

import jax
import jax.numpy as jnp
import numpy as np
from jax.experimental import pallas as pl
from jax.experimental.pallas import tpu as pltpu
from jax.experimental.pallas import tpu_sc as plsc

B, S, D = 2, 2048, 768
E, C, F = 64, 64, 3072
BS = B * S
BC = B * C
EBC = E * BC
NROWS = EBC + BS
SQ2PI = 0.7978845608028654

_HI = jax.lax.Precision.HIGHEST


def _fused_body(x_ref, wr_ref, br_ref, w1_ref, b1_ref, w2_ref,
                w1b_ref, b1b_ref, w2b_ref, b2_ref,
                y_ref, cmb_ref, dsp_v, dsp_s, scs_v, mp_v, xb_s, sem):
    i = pl.program_id(0)

    @pl.when(i == 0)
    def _router():
        logits = jnp.dot(x_ref[:], wr_ref[:],
                         preferred_element_type=jnp.float32) + br_ref[:]
        l3 = logits.reshape(B, S, E)
        m3 = jnp.max(l3, axis=-1, keepdims=True)
        ssum = jnp.sum(jnp.exp(l3 - m3), axis=-1, keepdims=True)
        mp3 = 1.0 / ssum
        ie = jax.lax.broadcasted_iota(jnp.int32, (B, S, E), 2)
        idx3 = jnp.min(jnp.where(l3 == m3, ie, E), axis=-1)
        oh = (ie == idx3[:, :, None]).astype(jnp.float32)
        cum = oh
        k = 1
        while k < S:
            cum = cum + jnp.concatenate(
                [jnp.zeros((B, k, E), jnp.float32), cum[:, :S - k, :]],
                axis=1)
            k *= 2
        posf = jnp.sum(cum * oh, axis=-1) - 1.0
        ic = jax.lax.broadcasted_iota(jnp.int32, (B, S, C), 2).astype(
            jnp.float32)
        poh = (ic == posf[:, :, None]).astype(jnp.float32)
        s1 = jax.lax.broadcasted_iota(jnp.int32, (B, S, E), 1).astype(
            jnp.float32) + 1.0
        dn = (((0,), (0,)), ((), ()))
        dsp_cols, sc_cols = [], []
        for b in range(B):
            stb = jax.lax.dot_general(oh[b] * s1[b], poh[b], dn,
                                      precision=_HI)
            scb = jax.lax.dot_general(oh[b] * mp3[b], poh[b], dn,
                                      precision=_HI)
            t = stb.astype(jnp.int32) - 1
            dsp_cols.append(jnp.maximum(t, 0) + b * S)
            sc_cols.append(scb)
        dsp_v[:] = jnp.concatenate(dsp_cols, axis=1)
        scs_v[:] = jnp.concatenate(sc_cols, axis=1)
        mp_v[:] = mp3.reshape(BS, 1)
        pos_i = posf.astype(jnp.int32)
        within = posf < float(C)
        bidx = jax.lax.broadcasted_iota(jnp.int32, (B, S), 0)
        sidx = jax.lax.broadcasted_iota(jnp.int32, (B, S), 1)
        slot_row = idx3 * BC + bidx * C + jnp.minimum(pos_i, C - 1)
        drop_row = EBC + bidx * S + sidx
        cmb_ref[:] = jnp.where(within, slot_row, drop_row)
        pltpu.make_async_copy(dsp_v, dsp_s, sem).start()
        pltpu.make_async_copy(dsp_v, dsp_s, sem).wait()

    @pl.when(i < E)
    def _expert():
        def gather(r, carry):
            t = dsp_s[i, r]
            xb_s[pl.ds(r, 1), :] = x_ref[pl.ds(t, 1), :]
            return carry

        jax.lax.fori_loop(0, BC, gather, 0, unroll=True)

        def half(w1h, b1h, w2h):
            h = jnp.dot(xb_s[:], w1h[0],
                        preferred_element_type=jnp.float32) + b1h[0]
            h = 0.5 * h * (1.0 + jnp.tanh(
                SQ2PI * (h + 0.044715 * (h * h * h))))
            return jnp.dot(h, w2h[0], preferred_element_type=jnp.float32)

        y = half(w1_ref, b1_ref, w2_ref) + half(w1b_ref, b1b_ref,
                                                w2b_ref) + b2_ref[0]
        s = scs_v[pl.ds(i, 1), :].reshape(BC)
        y_ref[:] = y * s[:, None]

    @pl.when(i >= E)
    def _init():
        base = (i - E) * BC
        y_ref[:] = x_ref[pl.ds(base, BC), :] * mp_v[pl.ds(base, BC), :]


def _fused(x2, Wr, br, W1, b1, W2, b2):
    ee = lambda i: jnp.minimum(i, E - 1)
    return pl.pallas_call(
        _fused_body,
        grid=(E + BS // BC,),
        in_specs=[
            pl.BlockSpec((BS, D), lambda i: (0, 0)),
            pl.BlockSpec((D, E), lambda i: (0, 0)),
            pl.BlockSpec((1, E), lambda i: (0, 0)),
            pl.BlockSpec((1, D, F // 2), lambda i: (ee(i), 0, 0)),
            pl.BlockSpec((1, 1, F // 2), lambda i: (ee(i), 0, 0)),
            pl.BlockSpec((1, F // 2, D), lambda i: (ee(i), 0, 0)),
            pl.BlockSpec((1, D, F // 2), lambda i: (ee(i), 0, 1)),
            pl.BlockSpec((1, 1, F // 2), lambda i: (ee(i), 0, 1)),
            pl.BlockSpec((1, F // 2, D), lambda i: (ee(i), 1, 0)),
            pl.BlockSpec((1, 1, D), lambda i: (ee(i), 0, 0)),
        ],
        out_specs=[
            pl.BlockSpec((BC, D), lambda i: (i, 0)),
            pl.BlockSpec((B, S), lambda i: (0, 0)),
        ],
        out_shape=[
            jax.ShapeDtypeStruct((NROWS, D), jnp.float32),
            jax.ShapeDtypeStruct((B, S), jnp.int32),
        ],
        scratch_shapes=[
            pltpu.VMEM((E, BC), jnp.int32),
            pltpu.SMEM((E, BC), jnp.int32),
            pltpu.VMEM((E, BC), jnp.float32),
            pltpu.VMEM((BS, 1), jnp.float32),
            pltpu.VMEM((BC, D), jnp.float32),
            pltpu.SemaphoreType.DMA,
        ],
        compiler_params=pltpu.CompilerParams(
            dimension_semantics=("arbitrary",),
            vmem_limit_bytes=64 * 1024 * 1024),
    )(x2, Wr, br.reshape(1, E), W1, b1.reshape(E, 1, F), W2,
      W1, b1.reshape(E, 1, F), W2, b2.reshape(E, 1, D))


def _sc_gather(table, idx, n_out):
    nw = 32
    b_per_w = n_out // nw
    ch = 128
    nch = b_per_w // ch
    mesh = plsc.VectorSubcoreMesh(core_axis_name="c", subcore_axis_name="s")

    @functools.partial(
        pl.kernel, mesh=mesh,
        out_type=jax.ShapeDtypeStruct((n_out, D), jnp.float32),
        scratch_types=[
            pltpu.VMEM((ch,), jnp.int32),
            pltpu.VMEM((ch, D), jnp.float32),
            pltpu.SemaphoreType.DMA,
        ],
    )
    def k(table_hbm, idx_hbm, out_hbm, idx_v, rows_v, sem):
        wid = jax.lax.axis_index("s") * 2 + jax.lax.axis_index("c")
        base = wid * b_per_w

        @pl.loop(0, nch)
        def _(j):
            off = base + j * ch
            pltpu.sync_copy(idx_hbm.at[pl.ds(off, ch)], idx_v)
            pltpu.async_copy(table_hbm.at[idx_v], rows_v, sem).wait()
            pltpu.sync_copy(rows_v, out_hbm.at[pl.ds(off, ch)])

    return k(table, idx)


def kernel(hidden_states, Wr, br, W1, b1, W2, b2):
    x2 = hidden_states.reshape(BS, D)
    ybig, cmb = _fused(x2, Wr, br, W1, b1, W2, b2)
    out = _sc_gather(ybig, cmb.reshape(BS), BS)
    return out.reshape(B, S, D)


# speedup vs baseline: 1.0124x; 1.0124x over previous
import functools

import jax
import jax.numpy as jnp
import numpy as np
from jax.experimental import pallas as pl
from jax.experimental.pallas import tpu as pltpu
from jax.experimental.pallas import tpu_sc as plsc

B, S, D = 2, 2048, 768
E, C, F = 64, 64, 3072
BS = B * S
BC = B * C
EBC = E * BC
NROWS = EBC + BS
SQ2PI = 0.7978845608028654

_HI = jax.lax.Precision.HIGHEST


def _fused_body(x_ref, wr_ref, br_ref, w1_ref, b1_ref, w2_ref, b2_ref,
                y_ref, cmb_ref, dsp_v, dsp_s, scs_v, mp_v, xb_s, sem):
    i = pl.program_id(0)

    @pl.when(i == 0)
    def _router():
        logits = jnp.dot(x_ref[:], wr_ref[:],
                         preferred_element_type=jnp.float32) + br_ref[:]
        l3 = logits.reshape(B, S, E)
        m3 = jnp.max(l3, axis=-1, keepdims=True)
        ssum = jnp.sum(jnp.exp(l3 - m3), axis=-1, keepdims=True)
        mp3 = 1.0 / ssum
        ie = jax.lax.broadcasted_iota(jnp.int32, (B, S, E), 2)
        idx3 = jnp.min(jnp.where(l3 == m3, ie, E), axis=-1)
        oh = (ie == idx3[:, :, None]).astype(jnp.float32)
        cum = oh
        k = 1
        while k < S:
            cum = cum + jnp.concatenate(
                [jnp.zeros((B, k, E), jnp.float32), cum[:, :S - k, :]],
                axis=1)
            k *= 2
        posf = jnp.sum(cum * oh, axis=-1) - 1.0
        ic = jax.lax.broadcasted_iota(jnp.int32, (B, S, C), 2).astype(
            jnp.float32)
        poh = (ic == posf[:, :, None]).astype(jnp.float32)
        s1 = jax.lax.broadcasted_iota(jnp.int32, (B, S, E), 1).astype(
            jnp.float32) + 1.0
        dn = (((0,), (0,)), ((), ()))
        dsp_cols, sc_cols = [], []
        for b in range(B):
            stb = jax.lax.dot_general(oh[b] * s1[b], poh[b], dn,
                                      precision=_HI)
            scb = jax.lax.dot_general(oh[b] * mp3[b], poh[b], dn,
                                      precision=_HI)
            t = stb.astype(jnp.int32) - 1
            dsp_cols.append(jnp.maximum(t, 0) + b * S)
            sc_cols.append(scb)
        dsp_v[:] = jnp.concatenate(dsp_cols, axis=1)
        scs_v[:] = jnp.concatenate(sc_cols, axis=1)
        mp_v[:] = mp3.reshape(BS, 1)
        pos_i = posf.astype(jnp.int32)
        within = posf < float(C)
        bidx = jax.lax.broadcasted_iota(jnp.int32, (B, S), 0)
        sidx = jax.lax.broadcasted_iota(jnp.int32, (B, S), 1)
        slot_row = idx3 * BC + bidx * C + jnp.minimum(pos_i, C - 1)
        drop_row = EBC + bidx * S + sidx
        cmb_ref[:] = jnp.where(within, slot_row, drop_row)
        pltpu.make_async_copy(dsp_v, dsp_s, sem).start()
        pltpu.make_async_copy(dsp_v, dsp_s, sem).wait()

    @pl.when(i < E)
    def _expert():
        def gather(r, carry):
            t = dsp_s[i, r]
            xb_s[pl.ds(r, 1), :] = x_ref[pl.ds(t, 1), :]
            return carry

        jax.lax.fori_loop(0, BC, gather, 0, unroll=True)
        h = jnp.dot(xb_s[:], w1_ref[0],
                    preferred_element_type=jnp.float32) + b1_ref[0]
        h = 0.5 * h * (1.0 + jnp.tanh(SQ2PI * (h + 0.044715 * (h * h * h))))
        y = jnp.dot(h, w2_ref[0],
                    preferred_element_type=jnp.float32) + b2_ref[0]
        s = scs_v[pl.ds(i, 1), :].reshape(BC)
        y_ref[:] = y * s[:, None]

    @pl.when(i >= E)
    def _init():
        base = (i - E) * BC
        y_ref[:] = x_ref[pl.ds(base, BC), :] * mp_v[pl.ds(base, BC), :]


def _fused(x2, Wr, br, W1, b1, W2, b2):
    ee = lambda i: jnp.minimum(i, E - 1)
    return pl.pallas_call(
        _fused_body,
        grid=(E + BS // BC,),
        in_specs=[
            pl.BlockSpec((BS, D), lambda i: (0, 0)),
            pl.BlockSpec((D, E), lambda i: (0, 0)),
            pl.BlockSpec((1, E), lambda i: (0, 0)),
            pl.BlockSpec((1, D, F), lambda i: (ee(i), 0, 0)),
            pl.BlockSpec((1, 1, F), lambda i: (ee(i), 0, 0)),
            pl.BlockSpec((1, F, D), lambda i: (ee(i), 0, 0)),
            pl.BlockSpec((1, 1, D), lambda i: (ee(i), 0, 0)),
        ],
        out_specs=[
            pl.BlockSpec((BC, D), lambda i: (i, 0)),
            pl.BlockSpec((B, S), lambda i: (0, 0)),
        ],
        out_shape=[
            jax.ShapeDtypeStruct((NROWS, D), jnp.float32),
            jax.ShapeDtypeStruct((B, S), jnp.int32),
        ],
        scratch_shapes=[
            pltpu.VMEM((E, BC), jnp.int32),
            pltpu.SMEM((E, BC), jnp.int32),
            pltpu.VMEM((E, BC), jnp.float32),
            pltpu.VMEM((BS, 1), jnp.float32),
            pltpu.VMEM((BC, D), jnp.float32),
            pltpu.SemaphoreType.DMA,
        ],
        compiler_params=pltpu.CompilerParams(
            dimension_semantics=("arbitrary",),
            vmem_limit_bytes=64 * 1024 * 1024),
    )(x2, Wr, br.reshape(1, E), W1, b1.reshape(E, 1, F), W2,
      b2.reshape(E, 1, D))


def _sc_gather(table, idx, n_out):
    nw = 32
    b_per_w = n_out // nw
    ch = 128
    nch = b_per_w // ch
    mesh = plsc.VectorSubcoreMesh(core_axis_name="c", subcore_axis_name="s")

    @functools.partial(
        pl.kernel, mesh=mesh,
        out_type=jax.ShapeDtypeStruct((n_out, D), jnp.float32),
        scratch_types=[
            pltpu.VMEM((ch,), jnp.int32),
            pltpu.VMEM((ch, D), jnp.float32),
            pltpu.SemaphoreType.DMA,
        ],
    )
    def k(table_hbm, idx_hbm, out_hbm, idx_v, rows_v, sem):
        wid = jax.lax.axis_index("s") * 2 + jax.lax.axis_index("c")
        base = wid * b_per_w

        @pl.loop(0, nch)
        def _(j):
            off = base + j * ch
            pltpu.sync_copy(idx_hbm.at[pl.ds(off, ch)], idx_v)
            pltpu.async_copy(table_hbm.at[idx_v], rows_v, sem).wait()
            pltpu.sync_copy(rows_v, out_hbm.at[pl.ds(off, ch)])

    return k(table, idx)


def kernel(hidden_states, Wr, br, W1, b1, W2, b2):
    x2 = hidden_states.reshape(BS, D)
    ybig, cmb = _fused(x2, Wr, br, W1, b1, W2, b2)
    out = _sc_gather(ybig, cmb.reshape(BS), BS)
    return out.reshape(B, S, D)
